# baseline (speedup 1.0000x reference)
import functools

import jax
import jax.numpy as jnp
from jax import lax
from jax.experimental import pallas as pl
from jax.experimental.pallas import tpu as pltpu
from jax.experimental.pallas import tpu_sc as plsc

_VOCAB = 100000
_D = 128
_MAX_LEN = 4096
_B = 1024
_L = 200

_NC, _NS = 2, 16
_NW = _NC * _NS
_N = _B * _L
_PER_W = _N // _NW
_CHUNK = _L
_NCHUNK = _PER_W // _CHUNK


def _pe_table():
    position = jnp.arange(_L)[:, None]
    i = jnp.arange(_D)[None, :]
    angles = position * (1.0 / jnp.power(10000.0, 2 * (i // 2) / _D))
    pe = jnp.zeros((_L, _D), dtype=jnp.float32)
    pe = pe.at[:, 0::2].set(jnp.sin(angles[:, 0::2]).astype(jnp.float32))
    pe = pe.at[:, 1::2].set(jnp.cos(angles[:, 1::2]).astype(jnp.float32))
    return pe


@functools.cache
def _build_sc_embed():
    @functools.partial(
        pl.kernel,
        out_type=jax.ShapeDtypeStruct((_N, _D), jnp.float32),
        mesh=plsc.VectorSubcoreMesh(
            core_axis_name="c", subcore_axis_name="s", num_cores=_NC, num_subcores=_NS
        ),
        scratch_types=[
            pltpu.VMEM((_PER_W,), jnp.int32),
            pltpu.VMEM((_CHUNK, _D), jnp.float32),
            pltpu.VMEM((_CHUNK, _D), jnp.float32),
            pltpu.VMEM((_CHUNK, _D), jnp.float32),
            pltpu.VMEM((_CHUNK, _D), jnp.float32),
            pltpu.SemaphoreType.DMA,
            pltpu.SemaphoreType.DMA,
            pltpu.SemaphoreType.DMA,
            pltpu.SemaphoreType.DMA,
            pltpu.SemaphoreType.DMA,
            pltpu.SemaphoreType.DMA,
            pltpu.SemaphoreType.DMA,
            pltpu.SemaphoreType.DMA,
            pltpu.SemaphoreType.DMA,
        ],
    )
    def _sc_embed(
        table_hbm, ids_hbm, pe_hbm, out_hbm,
        idx_v, rows0, rows1, rows2, pe_v,
        ga0, ga1, ga2, gb0, gb1, gb2, w0, w1, w2,
    ):
        wid = lax.axis_index("s") * _NC + lax.axis_index("c")
        base = wid * _PER_W
        pltpu.sync_copy(ids_hbm.at[pl.ds(base, _PER_W)], idx_v)
        bufs = (rows0, rows1, rows2)
        gasems = (ga0, ga1, ga2)
        gbsems = (gb0, gb1, gb2)
        wsems = (w0, w1, w2)

        def gather_parts(k, b):
            p1 = pltpu.make_async_copy(
                table_hbm.at[idx_v.at[pl.ds(k * _CHUNK, 104)]],
                bufs[b].at[pl.ds(0, 104)],
                gasems[b],
            )
            p2 = pltpu.make_async_copy(
                table_hbm.at[idx_v.at[pl.ds(k * _CHUNK + 104, 96)]],
                bufs[b].at[pl.ds(104, 96)],
                gbsems[b],
            )
            return p1, p2

        def start_gather(k, b):
            p1, p2 = gather_parts(k, b)
            p1.start()
            p2.start()

        def write_parts(k, b):
            p1 = pltpu.make_async_copy(
                bufs[b].at[pl.ds(0, 104)],
                out_hbm.at[pl.ds(base + k * _CHUNK, 104)],
                wsems[b],
            )
            p2 = pltpu.make_async_copy(
                bufs[b].at[pl.ds(104, 96)],
                out_hbm.at[pl.ds(base + k * _CHUNK + 104, 96)],
                wsems[b],
            )
            return p1, p2

        def start_write(k, b):
            p1, p2 = write_parts(k, b)
            p1.start()
            p2.start()

        def wait_write(k, b):
            p1, p2 = write_parts(k, b)
            p1.wait()
            p2.wait()

        def add_pe(b, lo, n):
            buf = bufs[b]

            @plsc.parallel_loop(lo, lo + n, step=2, unroll=2)
            def _(i):
                for r in range(2):
                    for c in range(_D // 16):
                        plsc.addupdate(
                            buf.at[i + r, pl.ds(c * 16, 16)],
                            pe_v[i + r, pl.ds(c * 16, 16)],
                        )

        def step(j, b, drain_prev, next_gather):
            p1, p2 = gather_parts(j, b)
            p1.wait()
            p2.wait()
            add_pe(b, 0, _CHUNK)
            start_write(j, b)
            pb = (b + 2) % 3
            if drain_prev:
                wait_write(j - 1, pb)
            if next_gather:
                start_gather(j + 2, pb)

        start_gather(0, 0)
        start_gather(1, 1)
        pltpu.sync_copy(pe_hbm, pe_v)
        step(0, 0, False, True)
        step(1, 1, True, True)
        step(2, 2, True, True)

        def ring_body(m, _):
            for b in range(3):
                step(3 * m + b, b, True, True)
            return 0

        lax.fori_loop(1, _NCHUNK // 3, ring_body, 0)

        step(30, 0, True, False)
        step(31, 1, True, False)
        wait_write(31, 1)

    return _sc_embed


def kernel(input_ids, embedding_table):
    flat_ids = input_ids.reshape(-1).astype(jnp.int32)
    pe = _pe_table()
    out = _build_sc_embed()(embedding_table, flat_ids, pe)
    return out.reshape(_B, _L, _D)


# speedup vs baseline: 1.0067x; 1.0067x over previous
import functools

import jax
import jax.numpy as jnp
from jax import lax
from jax.experimental import pallas as pl
from jax.experimental.pallas import tpu as pltpu
from jax.experimental.pallas import tpu_sc as plsc

_VOCAB = 100000
_D = 128
_MAX_LEN = 4096
_B = 1024
_L = 200

_NC, _NS = 2, 16
_NW = _NC * _NS
_N = _B * _L
_PER_W = _N // _NW
_CHUNK = _L
_NCHUNK = _PER_W // _CHUNK


def _pe_table():
    position = jnp.arange(_L)[:, None]
    i = jnp.arange(_D)[None, :]
    angles = position * (1.0 / jnp.power(10000.0, 2 * (i // 2) / _D))
    pe = jnp.zeros((_L, _D), dtype=jnp.float32)
    pe = pe.at[:, 0::2].set(jnp.sin(angles[:, 0::2]).astype(jnp.float32))
    pe = pe.at[:, 1::2].set(jnp.cos(angles[:, 1::2]).astype(jnp.float32))
    return pe


@functools.cache
def _build_sc_embed():
    @functools.partial(
        pl.kernel,
        out_type=jax.ShapeDtypeStruct((_N, _D), jnp.float32),
        mesh=plsc.VectorSubcoreMesh(
            core_axis_name="c", subcore_axis_name="s", num_cores=_NC, num_subcores=_NS
        ),
        scratch_types=[
            pltpu.VMEM((_PER_W,), jnp.int32),
            pltpu.VMEM((_CHUNK, _D), jnp.float32),
            pltpu.VMEM((_CHUNK, _D), jnp.float32),
            pltpu.VMEM((_CHUNK, _D), jnp.float32),
            pltpu.VMEM((_CHUNK, _D), jnp.float32),
            pltpu.SemaphoreType.DMA,
            pltpu.SemaphoreType.DMA,
            pltpu.SemaphoreType.DMA,
            pltpu.SemaphoreType.DMA,
            pltpu.SemaphoreType.DMA,
            pltpu.SemaphoreType.DMA,
            pltpu.SemaphoreType.DMA,
            pltpu.SemaphoreType.DMA,
            pltpu.SemaphoreType.DMA,
        ],
    )
    def _sc_embed(
        table_hbm, ids_hbm, pe_hbm, out_hbm,
        idx_v, rows0, rows1, rows2, pe_v,
        ga0, ga1, ga2, gb0, gb1, gb2, w0, w1, w2,
    ):
        wid = lax.axis_index("s") * _NC + lax.axis_index("c")
        base = wid * _PER_W
        pltpu.sync_copy(ids_hbm.at[pl.ds(base, _PER_W)], idx_v)
        bufs = (rows0, rows1, rows2)
        gasems = (ga0, ga1, ga2)
        gbsems = (gb0, gb1, gb2)
        wsems = (w0, w1, w2)

        def gather_parts(k, b):
            p1 = pltpu.make_async_copy(
                table_hbm.at[idx_v.at[pl.ds(k * _CHUNK, 104)]],
                bufs[b].at[pl.ds(0, 104)],
                gasems[b],
            )
            p2 = pltpu.make_async_copy(
                table_hbm.at[idx_v.at[pl.ds(k * _CHUNK + 104, 96)]],
                bufs[b].at[pl.ds(104, 96)],
                gbsems[b],
            )
            return p1, p2

        def start_gather(k, b):
            p1, p2 = gather_parts(k, b)
            p1.start()
            p2.start()

        def write_desc(k, b):
            return pltpu.make_async_copy(
                bufs[b], out_hbm.at[pl.ds(base + k * _CHUNK, _CHUNK)], wsems[b]
            )

        def start_write(k, b):
            write_desc(k, b).start()

        def wait_write(k, b):
            write_desc(k, b).wait()

        def add_pe(b, lo, n):
            buf = bufs[b]

            @plsc.parallel_loop(lo, lo + n, step=2, unroll=2)
            def _(i):
                for r in range(2):
                    for c in range(_D // 16):
                        plsc.addupdate(
                            buf.at[i + r, pl.ds(c * 16, 16)],
                            pe_v[i + r, pl.ds(c * 16, 16)],
                        )

        def step(j, b, drain_prev, next_gather):
            p1, p2 = gather_parts(j, b)
            p1.wait()
            p2.wait()
            add_pe(b, 0, _CHUNK)
            start_write(j, b)
            pb = (b + 2) % 3
            if drain_prev:
                wait_write(j - 1, pb)
            if next_gather:
                start_gather(j + 2, pb)

        start_gather(0, 0)
        start_gather(1, 1)
        pltpu.sync_copy(pe_hbm, pe_v)
        step(0, 0, False, True)
        step(1, 1, True, True)
        step(2, 2, True, True)

        def ring_body(m, _):
            for b in range(3):
                step(3 * m + b, b, True, True)
            return 0

        lax.fori_loop(1, _NCHUNK // 3, ring_body, 0)

        step(30, 0, True, False)
        step(31, 1, True, False)
        wait_write(31, 1)

    return _sc_embed


def kernel(input_ids, embedding_table):
    flat_ids = input_ids.reshape(-1).astype(jnp.int32)
    pe = _pe_table()
    out = _build_sc_embed()(embedding_table, flat_ids, pe)
    return out.reshape(_B, _L, _D)
